# Initial kernel scaffold; baseline (speedup 1.0000x reference)
import jax
import jax.numpy as jnp
from jax.experimental import pallas as pl


def kernel(thought_ids, emb_table, W, b):
    raise NotImplementedError("write your pallas kernel here")


# speedup vs baseline: 2.0819x; 2.0819x over previous
import functools

import jax
import jax.numpy as jnp
from jax import lax
from jax.experimental import pallas as pl
from jax.experimental.pallas import tpu as pltpu
from jax.experimental.pallas import tpu_sc as plsc

H = 128
T = 200
CHUNK = 8
L = 16
NC, NS = 2, 16
NW = NC * NS
GCH = 40
NGCH = T // GCH
HV = H // L


def _pool_body(ids_hbm, table_hbm, out_hbm, idx_v, rows_v, acc_v, sem0, sem1):
    bpw = ids_hbm.shape[0] // NW
    wid = lax.axis_index("s") * NC + lax.axis_index("c")
    base = wid * bpw

    pltpu.sync_copy(ids_hbm.at[pl.ds(base, bpw)], idx_v)

    def fire(b, buf, sem):
        for c in range(NGCH):
            pltpu.async_copy(
                table_hbm.at[idx_v.at[b, pl.ds(c * GCH, GCH)]],
                rows_v.at[buf, pl.ds(c * GCH, GCH)],
                sem,
            )

    def drain(buf, sem):
        for c in range(NGCH):
            pltpu.make_async_copy(
                table_hbm.at[pl.ds(0, GCH)],
                rows_v.at[buf, pl.ds(c * GCH, GCH)],
                sem,
            ).wait()

    def accum(b, buf):
        def body(t, accs):
            return tuple(
                accs[h] + rows_v[buf, t, pl.ds(h * L, L)] for h in range(HV)
            )
        accs = tuple(jnp.zeros((L,), jnp.float32) for _ in range(HV))
        accs = lax.fori_loop(0, T, body, accs, unroll=2)
        for h in range(HV):
            acc_v[b, pl.ds(h * L, L)] = accs[h] * (1.0 / T)

    fire(0, 0, sem0)

    def step(i, _):
        b0 = 2 * i
        fire(b0 + 1, 1, sem1)
        drain(0, sem0)
        accum(b0, 0)
        fire(jnp.minimum(b0 + 2, bpw - 1), 0, sem0)
        drain(1, sem1)
        accum(b0 + 1, 1)
        return 0

    lax.fori_loop(0, bpw // 2, step, 0)
    drain(0, sem0)

    pltpu.sync_copy(acc_v, out_hbm.at[pl.ds(base, bpw)])


def _pooled(thought_ids, emb_table):
    batch = thought_ids.shape[0]
    bpw = batch // NW
    mesh = plsc.VectorSubcoreMesh(
        core_axis_name="c", subcore_axis_name="s", num_cores=NC, num_subcores=NS
    )
    f = functools.partial(
        pl.kernel,
        mesh=mesh,
        compiler_params=pltpu.CompilerParams(use_tc_tiling_on_sc=False),
        out_type=jax.ShapeDtypeStruct((batch, H), jnp.float32),
        scratch_types=[
            pltpu.VMEM((bpw, T), jnp.int32),
            pltpu.VMEM((2, T, H), jnp.float32),
            pltpu.VMEM((bpw, H), jnp.float32),
            pltpu.SemaphoreType.DMA,
            pltpu.SemaphoreType.DMA,
        ],
    )(_pool_body)
    return f(thought_ids, emb_table)


def _proj_body(x_ref, wt_ref, b_ref, o_ref):
    o_ref[...] = (
        jnp.dot(x_ref[...], wt_ref[...], preferred_element_type=jnp.float32)
        + b_ref[...]
    )


def _proj(pooled, wt, bias):
    batch = pooled.shape[0]
    bm = 512
    grid = (batch // bm,)
    return pl.pallas_call(
        _proj_body,
        grid=grid,
        in_specs=[
            pl.BlockSpec((bm, H), lambda i: (i, 0)),
            pl.BlockSpec((H, H * CHUNK), lambda i: (0, 0)),
            pl.BlockSpec((1, H * CHUNK), lambda i: (0, 0)),
        ],
        out_specs=pl.BlockSpec((bm, H * CHUNK), lambda i: (i, 0)),
        out_shape=jax.ShapeDtypeStruct((batch, H * CHUNK), jnp.float32),
    )(pooled, wt, bias)


def kernel(thought_ids, emb_table, W, b):
    pooled = _pooled(thought_ids, emb_table)
    state = _proj(pooled, W.T, b.reshape(1, -1))
    return state.reshape(-1, CHUNK, H)


# speedup vs baseline: 2.0838x; 1.0009x over previous
import functools

import jax
import jax.numpy as jnp
from jax import lax
from jax.experimental import pallas as pl
from jax.experimental.pallas import tpu as pltpu
from jax.experimental.pallas import tpu_sc as plsc

H = 128
T = 200
CHUNK = 8
L = 16
NC, NS = 2, 16
NW = NC * NS
GCHUNKS = ((0, 128), (128, 72))
HV = H // L


def _pool_body(ids_hbm, table_hbm, out_hbm, idx_v, rows_v, acc_v, sem0, sem1):
    bpw = ids_hbm.shape[0] // NW
    wid = lax.axis_index("s") * NC + lax.axis_index("c")
    base = wid * bpw

    pltpu.sync_copy(ids_hbm.at[pl.ds(base, bpw)], idx_v)

    def fire(b, buf, sem):
        for off, ln in GCHUNKS:
            pltpu.async_copy(
                table_hbm.at[idx_v.at[b, pl.ds(off, ln)]],
                rows_v.at[buf, pl.ds(off, ln)],
                sem,
            )

    def drain(buf, sem):
        for off, ln in GCHUNKS:
            pltpu.make_async_copy(
                table_hbm.at[pl.ds(0, ln)],
                rows_v.at[buf, pl.ds(off, ln)],
                sem,
            ).wait()

    def accum(b, buf):
        def body(t, accs):
            return tuple(
                accs[h] + rows_v[buf, t, pl.ds(h * L, L)] for h in range(HV)
            )
        accs = tuple(jnp.zeros((L,), jnp.float32) for _ in range(HV))
        accs = lax.fori_loop(0, T, body, accs, unroll=2)
        for h in range(HV):
            acc_v[b, pl.ds(h * L, L)] = accs[h] * (1.0 / T)

    fire(0, 0, sem0)

    def step(i, _):
        b0 = 2 * i
        fire(b0 + 1, 1, sem1)
        drain(0, sem0)
        accum(b0, 0)
        fire(jnp.minimum(b0 + 2, bpw - 1), 0, sem0)
        drain(1, sem1)
        accum(b0 + 1, 1)
        return 0

    lax.fori_loop(0, bpw // 2, step, 0)
    drain(0, sem0)

    pltpu.sync_copy(acc_v, out_hbm.at[pl.ds(base, bpw)])


def _pooled(thought_ids, emb_table):
    batch = thought_ids.shape[0]
    bpw = batch // NW
    mesh = plsc.VectorSubcoreMesh(
        core_axis_name="c", subcore_axis_name="s", num_cores=NC, num_subcores=NS
    )
    f = functools.partial(
        pl.kernel,
        mesh=mesh,
        compiler_params=pltpu.CompilerParams(use_tc_tiling_on_sc=False),
        out_type=jax.ShapeDtypeStruct((batch, H), jnp.float32),
        scratch_types=[
            pltpu.VMEM((bpw, T), jnp.int32),
            pltpu.VMEM((2, T, H), jnp.float32),
            pltpu.VMEM((bpw, H), jnp.float32),
            pltpu.SemaphoreType.DMA,
            pltpu.SemaphoreType.DMA,
        ],
    )(_pool_body)
    return f(thought_ids, emb_table)


def _proj_body(x_ref, wt_ref, b_ref, o_ref):
    o_ref[...] = (
        jnp.dot(x_ref[...], wt_ref[...], preferred_element_type=jnp.float32)
        + b_ref[...]
    )


def _proj(pooled, wt, bias):
    batch = pooled.shape[0]
    bm = 512
    grid = (batch // bm,)
    return pl.pallas_call(
        _proj_body,
        grid=grid,
        in_specs=[
            pl.BlockSpec((bm, H), lambda i: (i, 0)),
            pl.BlockSpec((H, H * CHUNK), lambda i: (0, 0)),
            pl.BlockSpec((1, H * CHUNK), lambda i: (0, 0)),
        ],
        out_specs=pl.BlockSpec((bm, H * CHUNK), lambda i: (i, 0)),
        out_shape=jax.ShapeDtypeStruct((batch, H * CHUNK), jnp.float32),
    )(pooled, wt, bias)


def kernel(thought_ids, emb_table, W, b):
    pooled = _pooled(thought_ids, emb_table)
    state = _proj(pooled, W.T, b.reshape(1, -1))
    return state.reshape(-1, CHUNK, H)
